# baseline (speedup 1.0000x reference)
import functools

import jax
import jax.numpy as jnp
from jax import lax
from jax.experimental import pallas as pl
from jax.experimental.pallas import tpu as pltpu
from jax.experimental.pallas import tpu_sc as plsc

DIM = 64
M_DIM = 16
K = 32
CPAD = 16

T1 = 256
T3 = 256

INT_MAX = 2147483647
IDX_MASK = 2047
DIST_MASK = -2048


def _silu(x):
    return x * (0.5 * jnp.tanh(0.5 * x) + 0.5)


def _topk_body(n_nodes, coors_ref, coors_t_ref, idx_ref):
    b = pl.program_id(0)
    x = coors_ref[0]
    d = None
    for c in range(3):
        xi = x[:, c:c + 1]
        xj = coors_t_ref[0, c:c + 1, :]
        diff = xi - xj
        sq = diff * diff
        d = sq if d is None else d + sq
    d = d + 1.0
    bits = lax.bitcast_convert_type(d, jnp.int32)
    cols = lax.broadcasted_iota(jnp.int32, d.shape, 1)
    keys = lax.bitcast_convert_type(
        jnp.bitwise_or(jnp.bitwise_and(bits, DIST_MASK), cols), jnp.float32)
    big = jnp.float32(3.0e38)
    picked = []
    for _ in range(K):
        kmin = jnp.min(keys, axis=1, keepdims=True)
        picked.append(kmin)
        keys = jnp.where(keys == kmin, big, keys)
    allk = lax.bitcast_convert_type(
        jnp.concatenate(picked, axis=1), jnp.int32)
    idx_ref[0] = jnp.bitwise_and(allk, IDX_MASK) + b * n_nodes


def _topk_call(coors, coors_t):
    bsz, n, _ = coors.shape
    grid = (bsz, n // T1)
    return pl.pallas_call(
        functools.partial(_topk_body, n),
        grid=grid,
        in_specs=[
            pl.BlockSpec((1, T1, 3), lambda b, t: (b, t, 0)),
            pl.BlockSpec((1, 3, n), lambda b, t: (b, 0, 0)),
        ],
        out_specs=pl.BlockSpec((1, T1, K), lambda b, t: (b, t, 0)),
        out_shape=jax.ShapeDtypeStruct((bsz, n, K), jnp.int32),
    )(coors, coors_t)


_CHUNK = 512
TBL_W = 128


def _gather_body(n_per_worker, num_cores, tbl_ref, idx_ref, out_ref,
                 idx_v, rows_v, sem):
    wid = lax.axis_index("s") * num_cores + lax.axis_index("c")
    base = wid * n_per_worker

    def chunk(i, carry):
        off = base + i * _CHUNK
        pltpu.sync_copy(idx_ref.at[pl.ds(off, _CHUNK)], idx_v)
        pltpu.async_copy(tbl_ref.at[idx_v], rows_v, sem).wait()
        pltpu.sync_copy(rows_v, out_ref.at[pl.ds(off, _CHUNK)])
        return carry

    lax.fori_loop(0, n_per_worker // _CHUNK, chunk, 0)


def _gather_call(tbl, flat_idx):
    total = flat_idx.shape[0]
    width = tbl.shape[1]
    info = plsc.get_sparse_core_info()
    nw = info.num_cores * info.num_subcores
    n_per_worker = total // nw
    mesh = plsc.VectorSubcoreMesh(core_axis_name="c", subcore_axis_name="s")
    kern = functools.partial(
        pl.kernel,
        mesh=mesh,
        out_type=jax.ShapeDtypeStruct((total, width), jnp.float32),
        scratch_types=[
            pltpu.VMEM((_CHUNK,), jnp.int32),
            pltpu.VMEM((_CHUNK, width), jnp.float32),
            pltpu.SemaphoreType.DMA,
        ],
    )(functools.partial(_gather_body, n_per_worker, info.num_cores))
    return kern(tbl, flat_idx)


def _mlp_body(feats_ref, coors_ref, gath_ref,
              we1_ref, be1_ref, we2_ref, be2_ref,
              wc1_ref, bc1_ref, wc2_ref, bc2_ref,
              wn1_ref, bn1_ref, wn2_ref, bn2_ref,
              node_ref, coors_out_ref):
    fi = feats_ref[0]
    ci = coors_ref[0]
    g = gath_ref[0, 0]
    fj = g[:, :DIM]
    cj = g[:, DIM:DIM + CPAD]

    we1 = we1_ref[...]
    ei = jnp.dot(fi, we1[:DIM], preferred_element_type=jnp.float32) \
        + be1_ref[...]
    eib = jnp.broadcast_to(ei[:, None, :], (T3, K, ei.shape[1]))
    eib = eib.reshape(T3 * K, ei.shape[1])

    cib = jnp.broadcast_to(ci[:, None, :], (T3, K, CPAD)).reshape(T3 * K, CPAD)
    rc = cib - cj
    rd = jnp.sum(rc * rc, axis=1, keepdims=True)

    fj_ext = jnp.concatenate([fj, rd], axis=1)
    pre1 = eib + jnp.dot(fj_ext, we1[DIM:], preferred_element_type=jnp.float32)
    h = _silu(pre1)
    m = _silu(jnp.dot(h, we2_ref[...], preferred_element_type=jnp.float32)
              + be2_ref[...])

    ch = _silu(jnp.dot(m, wc1_ref[...], preferred_element_type=jnp.float32)
               + bc1_ref[...])
    cw = jnp.dot(ch, wc2_ref[...], preferred_element_type=jnp.float32) \
        + bc2_ref[...]

    coors_out_ref[0] = jnp.sum((cw * rc).reshape(T3, K, CPAD), axis=1) + ci

    m_i = jnp.sum(m.reshape(T3, K, M_DIM), axis=1)
    nin = jnp.concatenate([fi, m_i], axis=1)
    nh = _silu(jnp.dot(nin, wn1_ref[...], preferred_element_type=jnp.float32)
               + bn1_ref[...])
    node_ref[0] = jnp.dot(nh, wn2_ref[...], preferred_element_type=jnp.float32) \
        + bn2_ref[...] + fi


def _mlp_call(feats, coors16, gath4, W_e1, b_e1, W_e2, b_e2,
              W_c1, b_c1, W_c2, b_c2, W_n1, b_n1, W_n2, b_n2):
    bsz, n, _ = feats.shape
    grid = (bsz, n // T3)

    def wspec(shape):
        nd = len(shape)
        return pl.BlockSpec(shape, lambda b, t, _nd=nd: (0,) * _nd)

    return pl.pallas_call(
        _mlp_body,
        grid=grid,
        in_specs=[
            pl.BlockSpec((1, T3, DIM), lambda b, t: (b, t, 0)),
            pl.BlockSpec((1, T3, CPAD), lambda b, t: (b, t, 0)),
            pl.BlockSpec((1, 1, T3 * K, TBL_W), lambda b, t: (b, t, 0, 0)),
            wspec(W_e1.shape), wspec(b_e1.shape),
            wspec(W_e2.shape), wspec(b_e2.shape),
            wspec(W_c1.shape), wspec(b_c1.shape),
            wspec(W_c2.shape), wspec(b_c2.shape),
            wspec(W_n1.shape), wspec(b_n1.shape),
            wspec(W_n2.shape), wspec(b_n2.shape),
        ],
        out_specs=[
            pl.BlockSpec((1, T3, DIM), lambda b, t: (b, t, 0)),
            pl.BlockSpec((1, T3, CPAD), lambda b, t: (b, t, 0)),
        ],
        out_shape=[
            jax.ShapeDtypeStruct((bsz, n, DIM), jnp.float32),
            jax.ShapeDtypeStruct((bsz, n, CPAD), jnp.float32),
        ],
    )(feats, coors16, gath4, W_e1, b_e1, W_e2, b_e2,
      W_c1, b_c1, W_c2, b_c2, W_n1, b_n1, W_n2, b_n2)


def kernel(feats, coors, W_e1, b_e1, W_e2, b_e2, W_c1, b_c1, W_c2, b_c2,
           W_n1, b_n1, W_n2, b_n2):
    bsz, n, d = feats.shape

    coors_t = jnp.transpose(coors, (0, 2, 1))
    coors16 = jnp.concatenate(
        [coors, jnp.zeros((bsz, n, CPAD - 3), jnp.float32)], axis=-1)

    gidx = _topk_call(coors, coors_t)
    flat_idx = gidx.reshape(bsz * n * K)

    tbl = jnp.concatenate(
        [feats.reshape(bsz * n, d), coors16.reshape(bsz * n, CPAD),
         jnp.zeros((bsz * n, TBL_W - d - CPAD), jnp.float32)], axis=1)
    gath = _gather_call(tbl, flat_idx)
    gath4 = gath.reshape(bsz, n // T3, T3 * K, TBL_W)

    node_out, coors16_out = _mlp_call(
        feats, coors16, gath4, W_e1, b_e1, W_e2, b_e2,
        W_c1, b_c1, W_c2, b_c2, W_n1, b_n1, W_n2, b_n2)
    return node_out, coors16_out[..., :3]


# speedup vs baseline: 1.2142x; 1.2142x over previous
import functools

import jax
import jax.numpy as jnp
from jax import lax
from jax.experimental import pallas as pl
from jax.experimental.pallas import tpu as pltpu
from jax.experimental.pallas import tpu_sc as plsc

DIM = 64
M_DIM = 16
K = 32
CPAD = 16

T1 = 256
T3 = 256

INT_MAX = 2147483647
IDX_MASK = 2047
DIST_MASK = -2048


def _silu(x):
    return x * (0.5 * jnp.tanh(0.5 * x) + 0.5)


def _topk_body(n_nodes, coors_ref, coors_t_ref, idx_ref):
    b = pl.program_id(0)
    x = coors_ref[0]
    d = None
    for c in range(3):
        xi = x[:, c:c + 1]
        xj = coors_t_ref[0, c:c + 1, :]
        diff = xi - xj
        sq = diff * diff
        d = sq if d is None else d + sq
    d = d + 1.0
    bits = lax.bitcast_convert_type(d, jnp.int32)
    cols = lax.broadcasted_iota(jnp.int32, d.shape, 1)
    keys = lax.bitcast_convert_type(
        jnp.bitwise_or(jnp.bitwise_and(bits, DIST_MASK), cols), jnp.float32)
    big = jnp.float32(3.0e38)
    picked = []
    for _ in range(K):
        kmin = jnp.min(keys, axis=1, keepdims=True)
        picked.append(kmin)
        keys = jnp.where(keys == kmin, big, keys)
    allk = lax.bitcast_convert_type(
        jnp.concatenate(picked, axis=1), jnp.int32)
    idx_ref[0] = jnp.bitwise_and(allk, IDX_MASK) + b * n_nodes


def _topk_call(coors, coors_t):
    bsz, n, _ = coors.shape
    grid = (bsz, n // T1)
    return pl.pallas_call(
        functools.partial(_topk_body, n),
        grid=grid,
        in_specs=[
            pl.BlockSpec((1, T1, 3), lambda b, t: (b, t, 0)),
            pl.BlockSpec((1, 3, n), lambda b, t: (b, 0, 0)),
        ],
        out_specs=pl.BlockSpec((1, T1, K), lambda b, t: (b, t, 0)),
        out_shape=jax.ShapeDtypeStruct((bsz, n, K), jnp.int32),
    )(coors, coors_t)


_CHUNK = 512
TBL_W = 128


def _gather_body(n_per_worker, num_cores, tbl_ref, idx_ref, out_ref,
                 idx_v, rows_v, sem):
    wid = lax.axis_index("s") * num_cores + lax.axis_index("c")
    base = wid * n_per_worker

    def chunk(i, carry):
        off = base + i * _CHUNK
        pltpu.sync_copy(idx_ref.at[pl.ds(off, _CHUNK)], idx_v)
        pltpu.async_copy(tbl_ref.at[idx_v], rows_v, sem).wait()
        pltpu.sync_copy(rows_v, out_ref.at[pl.ds(off, _CHUNK)])
        return carry

    lax.fori_loop(0, n_per_worker // _CHUNK, chunk, 0)


def _gather_call(tbl, flat_idx):
    total = flat_idx.shape[0]
    width = tbl.shape[1]
    info = plsc.get_sparse_core_info()
    nw = info.num_cores * info.num_subcores
    n_per_worker = total // nw
    mesh = plsc.VectorSubcoreMesh(core_axis_name="c", subcore_axis_name="s")
    kern = functools.partial(
        pl.kernel,
        mesh=mesh,
        out_type=jax.ShapeDtypeStruct((total, width), jnp.float32),
        scratch_types=[
            pltpu.VMEM((_CHUNK,), jnp.int32),
            pltpu.VMEM((_CHUNK, width), jnp.float32),
            pltpu.SemaphoreType.DMA,
        ],
    )(functools.partial(_gather_body, n_per_worker, info.num_cores))
    return kern(tbl, flat_idx)


def _mlp_body(feats_ref, coors_ref, gath_ref,
              we1_ref, be1_ref, we2_ref, be2_ref,
              wc1_ref, bc1_ref, wc2_ref, bc2_ref,
              wn1_ref, bn1_ref, wn2_ref, bn2_ref,
              node_ref, coors_out_ref):
    fi = feats_ref[0]
    ci = coors_ref[0]
    g = gath_ref[0, 0]
    fj = g[:, :DIM]
    cj = g[:, DIM:DIM + CPAD]

    we1 = we1_ref[...]
    ei = jnp.dot(fi, we1[:DIM], preferred_element_type=jnp.float32) \
        + be1_ref[...]
    eib = jnp.broadcast_to(ei[:, None, :], (T3, K, ei.shape[1]))
    eib = eib.reshape(T3 * K, ei.shape[1])

    cib = jnp.broadcast_to(ci[:, None, :], (T3, K, CPAD)).reshape(T3 * K, CPAD)
    rc = cib - cj
    rd = jnp.sum(rc * rc, axis=1, keepdims=True)

    pre1 = (eib
            + jnp.dot(fj, we1[DIM:2 * DIM], preferred_element_type=jnp.float32)
            + rd * we1[2 * DIM:2 * DIM + 1, :])
    h = _silu(pre1)
    m = _silu(jnp.dot(h, we2_ref[...], preferred_element_type=jnp.float32)
              + be2_ref[...])

    ch = _silu(jnp.dot(m, wc1_ref[...], preferred_element_type=jnp.float32)
               + bc1_ref[...])
    cw = jnp.dot(ch, wc2_ref[...], preferred_element_type=jnp.float32) \
        + bc2_ref[...]

    coors_out_ref[0] = jnp.sum((cw * rc).reshape(T3, K, CPAD), axis=1) + ci

    m_i = jnp.sum(m.reshape(T3, K, M_DIM), axis=1)
    nin = jnp.concatenate([fi, m_i], axis=1)
    nh = _silu(jnp.dot(nin, wn1_ref[...], preferred_element_type=jnp.float32)
               + bn1_ref[...])
    node_ref[0] = jnp.dot(nh, wn2_ref[...], preferred_element_type=jnp.float32) \
        + bn2_ref[...] + fi


def _mlp_call(feats, coors16, gath4, W_e1, b_e1, W_e2, b_e2,
              W_c1, b_c1, W_c2, b_c2, W_n1, b_n1, W_n2, b_n2):
    bsz, n, _ = feats.shape
    grid = (bsz, n // T3)

    def wspec(shape):
        nd = len(shape)
        return pl.BlockSpec(shape, lambda b, t, _nd=nd: (0,) * _nd)

    return pl.pallas_call(
        _mlp_body,
        grid=grid,
        in_specs=[
            pl.BlockSpec((1, T3, DIM), lambda b, t: (b, t, 0)),
            pl.BlockSpec((1, T3, CPAD), lambda b, t: (b, t, 0)),
            pl.BlockSpec((1, 1, T3 * K, TBL_W), lambda b, t: (b, t, 0, 0)),
            wspec(W_e1.shape), wspec(b_e1.shape),
            wspec(W_e2.shape), wspec(b_e2.shape),
            wspec(W_c1.shape), wspec(b_c1.shape),
            wspec(W_c2.shape), wspec(b_c2.shape),
            wspec(W_n1.shape), wspec(b_n1.shape),
            wspec(W_n2.shape), wspec(b_n2.shape),
        ],
        out_specs=[
            pl.BlockSpec((1, T3, DIM), lambda b, t: (b, t, 0)),
            pl.BlockSpec((1, T3, CPAD), lambda b, t: (b, t, 0)),
        ],
        out_shape=[
            jax.ShapeDtypeStruct((bsz, n, DIM), jnp.float32),
            jax.ShapeDtypeStruct((bsz, n, CPAD), jnp.float32),
        ],
    )(feats, coors16, gath4, W_e1, b_e1, W_e2, b_e2,
      W_c1, b_c1, W_c2, b_c2, W_n1, b_n1, W_n2, b_n2)


def kernel(feats, coors, W_e1, b_e1, W_e2, b_e2, W_c1, b_c1, W_c2, b_c2,
           W_n1, b_n1, W_n2, b_n2):
    bsz, n, d = feats.shape

    coors_t = jnp.transpose(coors, (0, 2, 1))
    coors16 = jnp.concatenate(
        [coors, jnp.zeros((bsz, n, CPAD - 3), jnp.float32)], axis=-1)

    gidx = _topk_call(coors, coors_t)
    flat_idx = gidx.reshape(bsz * n * K)

    tbl = jnp.concatenate(
        [feats.reshape(bsz * n, d), coors16.reshape(bsz * n, CPAD),
         jnp.zeros((bsz * n, TBL_W - d - CPAD), jnp.float32)], axis=1)
    gath = _gather_call(tbl, flat_idx)
    gath4 = gath.reshape(bsz, n // T3, T3 * K, TBL_W)

    node_out, coors16_out = _mlp_call(
        feats, coors16, gath4, W_e1, b_e1, W_e2, b_e2,
        W_c1, b_c1, W_c2, b_c2, W_n1, b_n1, W_n2, b_n2)
    return node_out, coors16_out[..., :3]


# speedup vs baseline: 1.3733x; 1.1311x over previous
import functools

import jax
import jax.numpy as jnp
from jax import lax
from jax.experimental import pallas as pl
from jax.experimental.pallas import tpu as pltpu
from jax.experimental.pallas import tpu_sc as plsc

DIM = 64
M_DIM = 16
K = 32
CPAD = 16

T1 = 256
T3 = 256

INT_MAX = 2147483647
IDX_MASK = 2047
DIST_MASK = -2048


def _silu(x):
    return x * (0.5 * jnp.tanh(0.5 * x) + 0.5)


def _topk_body(base, coors_ref, coors_t_ref, idx_ref):
    x = coors_ref[0]
    d = None
    for c in range(3):
        xi = x[:, c:c + 1]
        xj = coors_t_ref[0, c:c + 1, :]
        diff = xi - xj
        sq = diff * diff
        d = sq if d is None else d + sq
    d = d + 1.0
    bits = lax.bitcast_convert_type(d, jnp.int32)
    cols = lax.broadcasted_iota(jnp.int32, d.shape, 1)
    keys = lax.bitcast_convert_type(
        jnp.bitwise_or(jnp.bitwise_and(bits, DIST_MASK), cols), jnp.float32)
    big = jnp.float32(3.0e38)
    picked = []
    for _ in range(K):
        kmin = jnp.min(keys, axis=1, keepdims=True)
        picked.append(kmin)
        keys = jnp.where(keys == kmin, big, keys)
    allk = lax.bitcast_convert_type(
        jnp.concatenate(picked, axis=1), jnp.int32)
    idx_ref[0] = jnp.bitwise_and(allk, IDX_MASK) + base


def _topk_call(coors, coors_t, base):
    bsz, n, _ = coors.shape
    grid = (bsz, n // T1)
    return pl.pallas_call(
        functools.partial(_topk_body, base),
        grid=grid,
        in_specs=[
            pl.BlockSpec((1, T1, 3), lambda b, t: (b, t, 0)),
            pl.BlockSpec((1, 3, n), lambda b, t: (b, 0, 0)),
        ],
        out_specs=pl.BlockSpec((1, T1, K), lambda b, t: (b, t, 0)),
        out_shape=jax.ShapeDtypeStruct((bsz, n, K), jnp.int32),
    )(coors, coors_t)


_CHUNK = 512
TBL_W = 128


def _gather_body(n_per_worker, num_cores, tbl_ref, idx_ref, out_ref,
                 idx_v, rows_v, sem):
    wid = lax.axis_index("s") * num_cores + lax.axis_index("c")
    base = wid * n_per_worker

    def chunk(i, carry):
        off = base + i * _CHUNK
        pltpu.sync_copy(idx_ref.at[pl.ds(off, _CHUNK)], idx_v)
        pltpu.async_copy(tbl_ref.at[idx_v], rows_v, sem).wait()
        pltpu.sync_copy(rows_v, out_ref.at[pl.ds(off, _CHUNK)])
        return carry

    lax.fori_loop(0, n_per_worker // _CHUNK, chunk, 0)


def _gather_call(tbl, flat_idx):
    total = flat_idx.shape[0]
    width = tbl.shape[1]
    info = plsc.get_sparse_core_info()
    nw = info.num_cores * info.num_subcores
    n_per_worker = total // nw
    mesh = plsc.VectorSubcoreMesh(core_axis_name="c", subcore_axis_name="s")
    kern = functools.partial(
        pl.kernel,
        mesh=mesh,
        out_type=jax.ShapeDtypeStruct((total, width), jnp.float32),
        scratch_types=[
            pltpu.VMEM((_CHUNK,), jnp.int32),
            pltpu.VMEM((_CHUNK, width), jnp.float32),
            pltpu.SemaphoreType.DMA,
        ],
    )(functools.partial(_gather_body, n_per_worker, info.num_cores))
    return kern(tbl, flat_idx)


def _mlp_body(feats_ref, coors_ref, gath_ref,
              we1_ref, be1_ref, we2_ref, be2_ref,
              wc1_ref, bc1_ref, wc2_ref, bc2_ref,
              wn1_ref, bn1_ref, wn2_ref, bn2_ref,
              node_ref, coors_out_ref):
    fi = feats_ref[0]
    ci = coors_ref[0]
    g = gath_ref[0, 0]
    fj = g[:, :DIM]
    cj = g[:, DIM:DIM + CPAD]

    we1 = we1_ref[...]
    ei = jnp.dot(fi, we1[:DIM], preferred_element_type=jnp.float32) \
        + be1_ref[...]
    eib = jnp.broadcast_to(ei[:, None, :], (T3, K, ei.shape[1]))
    eib = eib.reshape(T3 * K, ei.shape[1])

    cib = jnp.broadcast_to(ci[:, None, :], (T3, K, CPAD)).reshape(T3 * K, CPAD)
    rc = cib - cj
    rd = jnp.sum(rc * rc, axis=1, keepdims=True)

    pre1 = (eib
            + jnp.dot(fj, we1[DIM:2 * DIM], preferred_element_type=jnp.float32)
            + rd * we1[2 * DIM:2 * DIM + 1, :])
    h = _silu(pre1)
    m = _silu(jnp.dot(h, we2_ref[...], preferred_element_type=jnp.float32)
              + be2_ref[...])

    ch = _silu(jnp.dot(m, wc1_ref[...], preferred_element_type=jnp.float32)
               + bc1_ref[...])
    cw = jnp.dot(ch, wc2_ref[...], preferred_element_type=jnp.float32) \
        + bc2_ref[...]

    coors_out_ref[0] = jnp.sum((cw * rc).reshape(T3, K, CPAD), axis=1) + ci

    m_i = jnp.sum(m.reshape(T3, K, M_DIM), axis=1)
    nin = jnp.concatenate([fi, m_i], axis=1)
    nh = _silu(jnp.dot(nin, wn1_ref[...], preferred_element_type=jnp.float32)
               + bn1_ref[...])
    node_ref[0] = jnp.dot(nh, wn2_ref[...], preferred_element_type=jnp.float32) \
        + bn2_ref[...] + fi


def _mlp_call(feats, coors16, gath4, W_e1, b_e1, W_e2, b_e2,
              W_c1, b_c1, W_c2, b_c2, W_n1, b_n1, W_n2, b_n2):
    bsz, n, _ = feats.shape
    grid = (bsz, n // T3)

    def wspec(shape):
        nd = len(shape)
        return pl.BlockSpec(shape, lambda b, t, _nd=nd: (0,) * _nd)

    return pl.pallas_call(
        _mlp_body,
        grid=grid,
        in_specs=[
            pl.BlockSpec((1, T3, DIM), lambda b, t: (b, t, 0)),
            pl.BlockSpec((1, T3, CPAD), lambda b, t: (b, t, 0)),
            pl.BlockSpec((1, 1, T3 * K, TBL_W), lambda b, t: (b, t, 0, 0)),
            wspec(W_e1.shape), wspec(b_e1.shape),
            wspec(W_e2.shape), wspec(b_e2.shape),
            wspec(W_c1.shape), wspec(b_c1.shape),
            wspec(W_c2.shape), wspec(b_c2.shape),
            wspec(W_n1.shape), wspec(b_n1.shape),
            wspec(W_n2.shape), wspec(b_n2.shape),
        ],
        out_specs=[
            pl.BlockSpec((1, T3, DIM), lambda b, t: (b, t, 0)),
            pl.BlockSpec((1, T3, CPAD), lambda b, t: (b, t, 0)),
        ],
        out_shape=[
            jax.ShapeDtypeStruct((bsz, n, DIM), jnp.float32),
            jax.ShapeDtypeStruct((bsz, n, CPAD), jnp.float32),
        ],
    )(feats, coors16, gath4, W_e1, b_e1, W_e2, b_e2,
      W_c1, b_c1, W_c2, b_c2, W_n1, b_n1, W_n2, b_n2)


def kernel(feats, coors, W_e1, b_e1, W_e2, b_e2, W_c1, b_c1, W_c2, b_c2,
           W_n1, b_n1, W_n2, b_n2):
    bsz, n, d = feats.shape

    coors_t = jnp.transpose(coors, (0, 2, 1))
    coors16 = jnp.concatenate(
        [coors, jnp.zeros((bsz, n, CPAD - 3), jnp.float32)], axis=-1)
    tbl = jnp.concatenate(
        [feats.reshape(bsz * n, d), coors16.reshape(bsz * n, CPAD),
         jnp.zeros((bsz * n, TBL_W - d - CPAD), jnp.float32)], axis=1)

    gidx = [_topk_call(coors[b:b + 1], coors_t[b:b + 1], b * n)
            for b in range(bsz)]
    gath = [_gather_call(tbl, gidx[b].reshape(n * K)) for b in range(bsz)]

    outs = [_mlp_call(feats[b:b + 1], coors16[b:b + 1],
                      gath[b].reshape(1, n // T3, T3 * K, TBL_W),
                      W_e1, b_e1, W_e2, b_e2, W_c1, b_c1, W_c2, b_c2,
                      W_n1, b_n1, W_n2, b_n2)
            for b in range(bsz)]
    node_out = jnp.concatenate([o[0] for o in outs], axis=0)
    coors16_out = jnp.concatenate([o[1] for o in outs], axis=0)
    return node_out, coors16_out[..., :3]
